# baseline (speedup 1.0000x reference)
import functools

import numpy as np
import jax
import jax.numpy as jnp
from jax import lax
from jax.experimental import pallas as pl
from jax.experimental.pallas import tpu as pltpu
from jax.experimental.pallas import tpu_sc as plsc

_R = 18
_H = _W = 37
_FLAT = _H * _W
_NPIX = 1027
_NIN = 1039
_NW = 16
_PER_W = 96
_PAD = _NW * _PER_W
_L = 16
_FULL_W = _FLAT // _PER_W
_TAIL = _FLAT - _FULL_W * _PER_W


def _build_src():
    src = np.full((_PAD,), _NPIX, np.int32)
    p = 0
    for q in range(-_R, _R + 1):
        for r in range(max(-_R, -q - _R), min(_R, -q + _R) + 1):
            src[(q + _R) * _W + (r + _R)] = p
            p += 1
    return src


_SRC = _build_src()

_mesh = plsc.VectorSubcoreMesh(
    core_axis_name="c", subcore_axis_name="s", num_cores=1
)


@functools.partial(
    pl.kernel,
    mesh=_mesh,
    out_type=jax.ShapeDtypeStruct((_FLAT,), jnp.float32),
    scratch_types=[
        pltpu.VMEM((_NIN,), jnp.float32),
        pltpu.VMEM((_PER_W,), jnp.int32),
        pltpu.VMEM((_PER_W,), jnp.float32),
        pltpu.SemaphoreType.DMA,
        pltpu.SemaphoreType.DMA,
    ],
    compiler_params=pltpu.CompilerParams(needs_layout_passes=False),
)
def _hexa_gather(hexa_hbm, src_hbm, out_hbm, hexa_v, idx_v, out_v, sem0, sem1):
    wid = lax.axis_index("s")
    base = wid * _PER_W
    cp_hexa = pltpu.async_copy(hexa_hbm, hexa_v, sem0)
    cp_idx = pltpu.async_copy(src_hbm.at[pl.ds(base, _PER_W)], idx_v, sem1)
    cp_hexa.wait()
    cp_idx.wait()
    zeros = jnp.zeros((_L,), jnp.float32)
    for c in range(_PER_W // _L):
        idx = idx_v[pl.ds(c * _L, _L)]
        vals = plsc.load_gather(hexa_v, [idx])
        out_v[pl.ds(c * _L, _L)] = jnp.where(idx < _NPIX, vals, zeros)
    @pl.when(wid < _FULL_W)
    def _():
        pltpu.sync_copy(out_v, out_hbm.at[pl.ds(base, _PER_W)])
    @pl.when(wid == _FULL_W)
    def _():
        pltpu.sync_copy(
            out_v.at[pl.ds(0, _TAIL)], out_hbm.at[pl.ds(base, _TAIL)]
        )


def kernel(hexa):
    flat = _hexa_gather(hexa, jnp.asarray(_SRC))
    return flat.reshape(_H, _W)
